# Initial kernel scaffold; baseline (speedup 1.0000x reference)
import jax
import jax.numpy as jnp
from jax.experimental import pallas as pl


def kernel(queries, keys, values, W):
    raise NotImplementedError("write your pallas kernel here")


# speedup vs baseline: 6.7779x; 6.7779x over previous
import functools

import jax
import jax.numpy as jnp
from jax import lax
from jax.experimental import pallas as pl
from jax.experimental.pallas import tpu as pltpu
from jax.experimental.pallas import tpu_sc as plsc

KN = 10
D = 16
NK = 1000000
NQ = 1024
G = 256
NB = 3968
KPAD = NB * G
NCHUNK = NB // 16
NW = 32
QPW = NQ // NW
NEG = -3.0e38


def _a_body(qaug_ref, kaug_ref, gm_ref, pack_ref):
    kaug = kaug_ref[...]
    q2x = qaug_ref[...]
    kb = kaug[:, 0:16]
    k2c = kaug[:, 16:17]
    st = lax.dot_general(kb, q2x, (((1,), (1,)), ((), ())),
                         preferred_element_type=jnp.float32)
    st = st - k2c
    gm_ref[0, 0, :] = jnp.max(st, axis=0)
    sel = jnp.concatenate(
        [2.0 * jnp.eye(16, dtype=jnp.float32),
         jnp.zeros((16, 1), jnp.float32)], axis=1)
    ktw = lax.dot_general(sel, kaug, (((1,), (1,)), ((), ())),
                          precision=lax.Precision.HIGHEST,
                          preferred_element_type=jnp.float32)
    ktw = ktw.astype(jnp.bfloat16).astype(jnp.float32)
    neg = jnp.concatenate(
        [jnp.zeros((1, 16), jnp.float32),
         jnp.full((1, 1), -1.0, jnp.float32)], axis=1)
    nk2 = lax.dot_general(neg, kaug, (((1,), (1,)), ((), ())),
                          precision=lax.Precision.HIGHEST,
                          preferred_element_type=jnp.float32)
    pack_ref[0, 0:16, :] = ktw
    pack_ref[0, 16:17, :] = nk2


def _run_a(qaug, kaug):
    return pl.pallas_call(
        _a_body,
        grid=(NB,),
        in_specs=[
            pl.BlockSpec((NQ, 16), lambda i: (0, 0)),
            pl.BlockSpec((G, 17), lambda i: (i, 0)),
        ],
        out_specs=[
            pl.BlockSpec((1, 1, NQ), lambda i: (i, 0, 0)),
            pl.BlockSpec((1, 24, G), lambda i: (i, 0, 0)),
        ],
        out_shape=[
            jax.ShapeDtypeStruct((NB, 1, NQ), jnp.float32),
            jax.ShapeDtypeStruct((NB, 24, G), jnp.float32),
        ],
        compiler_params=pltpu.CompilerParams(
            dimension_semantics=("arbitrary",)),
    )(qaug, kaug)


def _a2_body(gm_ref, gmt_ref):
    x = gm_ref[:, 0, :]
    eye = jnp.eye(128, dtype=jnp.float32)
    gmt_ref[...] = lax.dot_general(x, eye, (((0,), (0,)), ((), ())),
                                   precision=lax.Precision.HIGHEST,
                                   preferred_element_type=jnp.float32)


def _run_a2(gm):
    return pl.pallas_call(
        _a2_body,
        grid=(NB // 128,),
        in_specs=[pl.BlockSpec((128, 1, NQ), lambda i: (i, 0, 0))],
        out_specs=pl.BlockSpec((NQ, 128), lambda i: (0, i)),
        out_shape=jax.ShapeDtypeStruct((NQ, NB), jnp.float32),
        compiler_params=pltpu.CompilerParams(
            dimension_semantics=("arbitrary",)),
    )(gm)


def _merge_top16(rv, ri, v, idx):
    sv, si = plsc.sort_key_val(v, idx, descending=True)
    bv = lax.rev(sv, (0,))
    bi = lax.rev(si, (0,))
    keep = rv >= bv
    mv = jnp.where(keep, rv, bv)
    mi = jnp.where(keep, ri, bi)
    out = plsc.sort_key_val(mv, mi, descending=True)
    return out[0], out[1]


def _lane(vec, j):
    return jnp.max(jnp.where(lax.iota(jnp.int32, 16) == j, vec,
                             jnp.full((16,), NEG, vec.dtype)))


def _sc_body(gmt_hbm, pack_hbm, q_hbm, values_hbm, stacked_hbm,
             row_v, qv, gidx_v, packbuf, vidx_v, vrows, stackbuf, sem):
    wid = lax.axis_index("s") * 2 + lax.axis_index("c")

    def per_query(j, _):
        q = wid * QPW + j
        pltpu.sync_copy(gmt_hbm.at[q], row_v)
        pltpu.sync_copy(q_hbm.at[q], qv)

        def p1(c, carry):
            rv, ri = carry
            v = row_v[pl.ds(c * 16, 16)]
            idx = c * 16 + lax.iota(jnp.int32, 16)
            return _merge_top16(rv, ri, v, idx)

        rv0 = jnp.full((16,), NEG, jnp.float32)
        ri0 = jnp.zeros((16,), jnp.int32)
        rv, ri = lax.fori_loop(0, NCHUNK, p1, (rv0, ri0))
        gidx_v[...] = ri
        tau0 = _lane(rv, 9) - 0.05

        pltpu.async_copy(pack_hbm.at[gidx_v], packbuf, sem).wait()
        qvec = qv[...]
        qs = [qvec[f] for f in range(16)]

        def p2(c2, carry):
            rv2, ri2, tau = carry
            g = c2 // 16
            cc = c2 % 16
            sl = pl.ds(cc * 16, 16)
            acc = packbuf[g, 16, sl]
            for f in range(16):
                acc = acc + qs[f] * packbuf[g, f, sl]
            gidvec = plsc.load_gather(
                gidx_v, [jnp.full((16,), g, jnp.int32)])
            kidx = gidvec * G + cc * 16 + lax.iota(jnp.int32, 16)

            def do_merge(args):
                rv3, ri3 = _merge_top16(args[0], args[1], acc, kidx)
                return rv3, ri3, _lane(rv3, 9)

            return lax.cond(jnp.max(acc) > tau, do_merge,
                            lambda args: args, (rv2, ri2, tau))

        rv2, ri2, _ = lax.fori_loop(0, 16 * (G // 16), p2, (rv0, ri0, tau0))

        lanes = lax.iota(jnp.int32, 16)
        e = jnp.exp(rv2 - jnp.max(rv2))
        e = jnp.where(lanes < KN, e, 0.0)
        wn = e / jnp.sum(e)
        vidx_v[...] = ri2 // 8
        pltpu.async_copy(values_hbm.at[vidx_v], vrows, sem).wait()
        for k in range(KN):
            off = (ri2[k] % 8) * 16
            stackbuf[pl.ds(k * 16, 16)] = wn[k] * vrows[k, pl.ds(off, 16)]
        pltpu.sync_copy(stackbuf, stacked_hbm.at[q])
        return 0

    lax.fori_loop(0, QPW, per_query, 0)


def _run_sc(gmt, pack, queries, values):
    mesh = plsc.VectorSubcoreMesh(core_axis_name="c", subcore_axis_name="s")
    f = functools.partial(
        pl.kernel, _sc_body, mesh=mesh,
        compiler_params=pltpu.CompilerParams(needs_layout_passes=False),
        out_type=jax.ShapeDtypeStruct((NQ, KN * D), jnp.float32),
        scratch_types=[
            pltpu.VMEM((NB,), jnp.float32),
            pltpu.VMEM((16,), jnp.float32),
            pltpu.VMEM((16,), jnp.int32),
            pltpu.VMEM((16, 24, G), jnp.float32),
            pltpu.VMEM((16,), jnp.int32),
            pltpu.VMEM((16, 128), jnp.float32),
            pltpu.VMEM((KN * 16,), jnp.float32),
            pltpu.SemaphoreType.DMA,
        ],
    )()
    return f(gmt, pack, queries, values)


def _d_body(q_ref, s_ref, w_ref, o_ref):
    o_ref[...] = q_ref[...] + jnp.dot(s_ref[...], w_ref[...],
                                      preferred_element_type=jnp.float32)


def _run_d(queries, stacked, W):
    return pl.pallas_call(
        _d_body,
        out_shape=jax.ShapeDtypeStruct((NQ, D), jnp.float32),
    )(queries, stacked, W)


def kernel(queries, keys, values, W):
    keys_p = jnp.pad(keys, ((0, KPAD - NK), (0, 0)), constant_values=100.0)
    k2 = jnp.sum(keys_p * keys_p, axis=1, keepdims=True)
    kaug = jnp.concatenate([keys_p, k2], axis=1)
    qaug = 2.0 * queries
    qr = queries.astype(jnp.bfloat16).astype(jnp.float32)
    gm, pack = _run_a(qaug, kaug)
    gmt = _run_a2(gm)
    stacked = _run_sc(gmt, pack, qr, values.reshape(NK // 8, 128))
    return _run_d(queries, stacked, W)


# speedup vs baseline: 9.7307x; 1.4357x over previous
import functools

import jax
import jax.numpy as jnp
from jax import lax
from jax.experimental import pallas as pl
from jax.experimental.pallas import tpu as pltpu
from jax.experimental.pallas import tpu_sc as plsc

KN = 10
D = 16
NK = 1000000
NQ = 1024
G = 256
NB = 3968
KPAD = NB * G
NCHUNK = NB // 16
NW = 32
QPW = NQ // NW
GPS = 8
NEG = -3.0e38


def _a_body(q2x_ref, keys_ref, gm_ref, pack_ref):
    kb = keys_ref[...]
    q2x = q2x_ref[...]
    k2c = jnp.sum(kb * kb, axis=1, keepdims=True)
    st = lax.dot_general(kb, q2x, (((1,), (1,)), ((), ())),
                         preferred_element_type=jnp.float32)
    st = st - k2c
    sel = jnp.concatenate(
        [2.0 * jnp.eye(16, dtype=jnp.float32),
         jnp.zeros((16, 1), jnp.float32)], axis=1)
    for j in range(GPS):
        rows = st[j * G:(j + 1) * G, :]
        gm_ref[j, 0, :] = jnp.max(rows, axis=0)
        kbj = kb[j * G:(j + 1) * G, :]
        k2j = k2c[j * G:(j + 1) * G, :]
        kaugj = jnp.concatenate([kbj, k2j], axis=1)
        ktw = lax.dot_general(sel, kaugj, (((1,), (1,)), ((), ())),
                              precision=lax.Precision.HIGHEST,
                              preferred_element_type=jnp.float32)
        pack_ref[j, 0:16, :] = ktw.astype(jnp.bfloat16).astype(jnp.float32)
        neg = jnp.concatenate(
            [jnp.zeros((1, 16), jnp.float32),
             jnp.full((1, 1), -1.0, jnp.float32)], axis=1)
        nk2 = lax.dot_general(neg, kaugj, (((1,), (1,)), ((), ())),
                              precision=lax.Precision.HIGHEST,
                              preferred_element_type=jnp.float32)
        pack_ref[j, 16:17, :] = nk2


def _run_a(qaug, keys_p):
    return pl.pallas_call(
        _a_body,
        grid=(NB // GPS,),
        in_specs=[
            pl.BlockSpec((NQ, 16), lambda i: (0, 0)),
            pl.BlockSpec((GPS * G, 16), lambda i: (i, 0)),
        ],
        out_specs=[
            pl.BlockSpec((GPS, 1, NQ), lambda i: (i, 0, 0)),
            pl.BlockSpec((GPS, 24, G), lambda i: (i, 0, 0)),
        ],
        out_shape=[
            jax.ShapeDtypeStruct((NB, 1, NQ), jnp.float32),
            jax.ShapeDtypeStruct((NB, 24, G), jnp.float32),
        ],
        compiler_params=pltpu.CompilerParams(
            dimension_semantics=("arbitrary",)),
    )(qaug, keys_p)


def _a2_body(gm_ref, gmt_ref):
    x = gm_ref[:, 0, :]
    eye = jnp.eye(128, dtype=jnp.float32)
    gmt_ref[...] = lax.dot_general(x, eye, (((0,), (0,)), ((), ())),
                                   precision=lax.Precision.HIGHEST,
                                   preferred_element_type=jnp.float32)


def _run_a2(gm):
    return pl.pallas_call(
        _a2_body,
        grid=(NB // 128,),
        in_specs=[pl.BlockSpec((128, 1, NQ), lambda i: (i, 0, 0))],
        out_specs=pl.BlockSpec((NQ, 128), lambda i: (0, i)),
        out_shape=jax.ShapeDtypeStruct((NQ, NB), jnp.float32),
        compiler_params=pltpu.CompilerParams(
            dimension_semantics=("arbitrary",)),
    )(gm)


def _merge_top16(rv, ri, v, idx):
    sv, si = plsc.sort_key_val(v, idx, descending=True)
    bv = lax.rev(sv, (0,))
    bi = lax.rev(si, (0,))
    keep = rv >= bv
    mv = jnp.where(keep, rv, bv)
    mi = jnp.where(keep, ri, bi)
    out = plsc.sort_key_val(mv, mi, descending=True)
    return out[0], out[1]


def _lane(vec, j):
    return jnp.max(jnp.where(lax.iota(jnp.int32, 16) == j, vec,
                             jnp.full((16,), NEG, vec.dtype)))


def _sc_body(gmt_hbm, pack_hbm, q_hbm, values_hbm, stacked_hbm,
             row_v, qv, gidx_v, packbuf, vidx_v, vrows, stackbuf,
             sem):
    wid = lax.axis_index("s") * 2 + lax.axis_index("c")

    def per_query(j, _):
        q = wid * QPW + j
        pltpu.sync_copy(gmt_hbm.at[q], row_v)
        pltpu.sync_copy(q_hbm.at[q], qv)

        def p1(c, carry):
            rv, ri, tmin = carry
            v = row_v[pl.ds(c * 16, 16)]
            idx = c * 16 + lax.iota(jnp.int32, 16)

            def m(args):
                rv2, ri2 = _merge_top16(args[0], args[1], v, idx)
                return rv2, ri2, jnp.min(rv2)

            return lax.cond(jnp.max(v) > tmin, m, lambda a: a, carry)

        rv0 = jnp.full((16,), NEG, jnp.float32)
        ri0 = jnp.zeros((16,), jnp.int32)
        rv, ri, _ = lax.fori_loop(0, NCHUNK, p1, (rv0, ri0, NEG))
        gidx_v[...] = ri
        tau0 = _lane(rv, 9) - 0.05

        pltpu.async_copy(pack_hbm.at[gidx_v.at[pl.ds(0, KN)]],
                         packbuf, sem).wait()
        qvec = qv[...]
        qs = [qvec[f] for f in range(16)]

        def p2(c2, carry):
            rv2, ri2, tau = carry
            g = c2 // 16
            cc = c2 % 16
            sl = pl.ds(cc * 16, 16)
            acc = packbuf[g, 16, sl]
            for f in range(16):
                acc = acc + qs[f] * packbuf[g, f, sl]
            gidvec = plsc.load_gather(
                gidx_v, [jnp.full((16,), g, jnp.int32)])
            kidx = gidvec * G + cc * 16 + lax.iota(jnp.int32, 16)

            def do_merge(args):
                rv3, ri3 = _merge_top16(args[0], args[1], acc, kidx)
                return rv3, ri3, _lane(rv3, 9)

            return lax.cond(jnp.max(acc) > tau, do_merge,
                            lambda args: args, (rv2, ri2, tau))

        rv2, ri2, _ = lax.fori_loop(0, KN * (G // 16), p2, (rv0, ri0, tau0))

        lanes = lax.iota(jnp.int32, 16)
        e = jnp.exp(rv2 - jnp.max(rv2))
        e = jnp.where(lanes < KN, e, 0.0)
        wn = e / jnp.sum(e)
        vidx_v[...] = ri2 // 8
        pltpu.async_copy(values_hbm.at[vidx_v], vrows, sem).wait()
        for k in range(KN):
            off = (ri2[k] % 8) * 16
            stackbuf[pl.ds(k * 16, 16)] = wn[k] * vrows[k, pl.ds(off, 16)]
        pltpu.sync_copy(stackbuf, stacked_hbm.at[q])
        return 0

    lax.fori_loop(0, QPW, per_query, 0)


def _run_sc(gmt, pack, queries, values):
    mesh = plsc.VectorSubcoreMesh(core_axis_name="c", subcore_axis_name="s")
    f = functools.partial(
        pl.kernel, _sc_body, mesh=mesh,
        compiler_params=pltpu.CompilerParams(needs_layout_passes=False),
        out_type=jax.ShapeDtypeStruct((NQ, KN * D), jnp.float32),
        scratch_types=[
            pltpu.VMEM((NB,), jnp.float32),
            pltpu.VMEM((16,), jnp.float32),
            pltpu.VMEM((16,), jnp.int32),
            pltpu.VMEM((KN, 24, G), jnp.float32),
            pltpu.VMEM((16,), jnp.int32),
            pltpu.VMEM((16, 128), jnp.float32),
            pltpu.VMEM((KN * 16,), jnp.float32),
            pltpu.SemaphoreType.DMA,
        ],
    )()
    return f(gmt, pack, queries, values)


def _d_body(q_ref, s_ref, w_ref, o_ref):
    o_ref[...] = q_ref[...] + jnp.dot(s_ref[...], w_ref[...],
                                      preferred_element_type=jnp.float32)


def _run_d(queries, stacked, W):
    return pl.pallas_call(
        _d_body,
        out_shape=jax.ShapeDtypeStruct((NQ, D), jnp.float32),
    )(queries, stacked, W)


def kernel(queries, keys, values, W):
    keys_p = jnp.pad(keys, ((0, KPAD - NK), (0, 0)), constant_values=100.0)
    qaug = 2.0 * queries
    qr = queries.astype(jnp.bfloat16).astype(jnp.float32)
    gm, pack = _run_a(qaug, keys_p)
    gmt = _run_a2(gm)
    stacked = _run_sc(gmt, pack, qr, values.reshape(NK // 8, 128))
    return _run_d(queries, stacked, W)
